# Initial kernel scaffold; baseline (speedup 1.0000x reference)
import jax
import jax.numpy as jnp
from jax.experimental import pallas as pl


def kernel(high_dim_features, low_dim_features, edge_index, W_emb, b_emb, W1, att_src1, att_dst1, bias1, W2, att_src2, att_dst2, bias2):
    raise NotImplementedError("write your pallas kernel here")


# speedup vs baseline: 69.4457x; 69.4457x over previous
import functools

import jax
import jax.numpy as jnp
from jax import lax
from jax.experimental import pallas as pl
from jax.experimental.pallas import tpu as pltpu
from jax.experimental.pallas import tpu_sc as plsc

N1 = 10240
NCORES = 2
NSUB = 16
NTILES = NCORES * NSUB
BLK = 128


def _elu(x):
    return jnp.where(x > 0, x, jnp.exp(x) - 1.0)


RB = 1280


def _row_spec(cols):
    return pl.BlockSpec((RB, cols), lambda i: (i, 0))


def _full_spec(shape):
    nd = len(shape)
    return pl.BlockSpec(shape, lambda i: (0,) * nd)


def _tc_prologue(high_p, low_p, W_emb, b_emb, W1a, W1b, S2, D2):
    def body(h_ref, l_ref, we_ref, be_ref, w1a_ref, w1b_ref, s2_ref, d2_ref,
             tsrc_ref, tdst_ref):
        le = _elu(jnp.dot(l_ref[...], we_ref[...],
                          preferred_element_type=jnp.float32) + be_ref[...])
        h1 = (jnp.dot(h_ref[...], w1a_ref[...],
                      preferred_element_type=jnp.float32)
              + jnp.dot(le, w1b_ref[...], preferred_element_type=jnp.float32))
        tsrc_ref[:, 0:64] = h1
        tsrc_ref[:, 64:80] = jnp.dot(h1, s2_ref[...],
                                     preferred_element_type=jnp.float32)
        tdst_ref[...] = jnp.dot(h1, d2_ref[...],
                                preferred_element_type=jnp.float32)

    args = (high_p, low_p, W_emb, b_emb, W1a, W1b, S2, D2)
    return pl.pallas_call(
        body,
        grid=(N1 // RB,),
        in_specs=[_row_spec(high_p.shape[1]), _row_spec(low_p.shape[1])]
                 + [_full_spec(a.shape) for a in args[2:]],
        out_specs=(_row_spec(80), _row_spec(16)),
        out_shape=(jax.ShapeDtypeStruct((N1, 80), jnp.float32),
                   jax.ShapeDtypeStruct((N1, 16), jnp.float32)),
    )(*args)


def _tc_mid(acc_a, acc_b, R, bias1, W2, a2s, a2d):
    def body(a_ref, b_ref, r_ref, b1_ref, w2_ref, s_ref, d_ref,
             tsrc_ref, tdst_ref):
        s = a_ref[...] + b_ref[...]
        den = jnp.dot(s[:, 0:16], r_ref[...],
                      preferred_element_type=jnp.float32)
        x1 = _elu(s[:, 16:80] / (den + 1e-16) + b1_ref[...])
        h2 = jnp.dot(x1, w2_ref[...], preferred_element_type=jnp.float32)
        tsrc_ref[:, 0:16] = h2
        tsrc_ref[:, 16:32] = jnp.dot(h2, s_ref[...],
                                     preferred_element_type=jnp.float32)
        tdst_ref[...] = jnp.dot(h2, d_ref[...],
                                preferred_element_type=jnp.float32)

    args = (acc_a, acc_b, R, bias1, W2, a2s, a2d)
    return pl.pallas_call(
        body,
        grid=(N1 // RB,),
        in_specs=[_row_spec(80), _row_spec(80)]
                 + [_full_spec(a.shape) for a in args[2:]],
        out_specs=(_row_spec(32), _row_spec(16)),
        out_shape=(jax.ShapeDtypeStruct((N1, 32), jnp.float32),
                   jax.ShapeDtypeStruct((N1, 16), jnp.float32)),
    )(*args)


def _tc_epilogue(acc_a, acc_b, bias2):
    def body(a_ref, b_ref, b2_ref, out_ref):
        s = a_ref[...] + b_ref[...]
        o = s[:, 16:32] / (s[:, 0:1] + 1e-16) + b2_ref[...]
        m = jnp.max(o, axis=1, keepdims=True)
        z = o - m
        lse = jnp.log(jnp.sum(jnp.exp(z), axis=1, keepdims=True))
        out_ref[...] = z - lse

    return pl.pallas_call(
        body,
        grid=(N1 // RB,),
        in_specs=[_row_spec(32), _row_spec(32), _full_spec(bias2.shape)],
        out_specs=_row_spec(16),
        out_shape=jax.ShapeDtypeStruct((N1, 16), jnp.float32),
    )(acc_a, acc_b, bias2)


_GATHER_DNUMS = lax.GatherDimensionNumbers(
    offset_dims=(), collapsed_slice_dims=(0,), start_index_map=(0,))


def _vperm(x, idx):
    return lax.gather(x, idx[:, None], _GATHER_DNUMS, (1,),
                      mode=lax.GatherScatterMode.PROMISE_IN_BOUNDS)


def _edge_compute_l1(srows, drows, orows, e):
    sa = srows[e, pl.ds(64, 16)]
    da = drows[e, pl.ds(0, 16)]
    al = sa + da
    w = jnp.exp(jnp.maximum(al, 0.2 * al))
    orows[e, pl.ds(0, 16)] = w
    lane_half = lax.shift_right_logical(lax.iota(jnp.int32, 16), 3)
    for k in range(4):
        hk = srows[e, pl.ds(16 * k, 16)]
        wk = _vperm(w, lane_half + 2 * k)
        orows[e, pl.ds(16 + 16 * k, 16)] = hk * wk


def _edge_compute_l2(srows, drows, orows, e):
    sa = srows[e, pl.ds(16, 16)]
    da = drows[e, pl.ds(0, 16)]
    al = sa + da
    w = jnp.exp(jnp.maximum(al, 0.2 * al))
    orows[e, pl.ds(0, 16)] = w
    orows[e, pl.ds(16, 16)] = srows[e, pl.ds(0, 16)] * w


def _sc_edge_pass(tsrc, tdst, src_idx, dst_idx, nb, row_s, row_o, edge_fn):
    rows_per_sub = N1 // NSUB
    zchunk = 128
    mesh = plsc.VectorSubcoreMesh(core_axis_name="c", subcore_axis_name="s")

    @functools.partial(
        pl.kernel,
        mesh=mesh,
        compiler_params=pltpu.CompilerParams(use_tc_tiling_on_sc=False),
        out_type=jax.ShapeDtypeStruct((NCORES, N1, row_o), jnp.float32),
        scratch_types=[
            pltpu.VMEM((nb, BLK), jnp.int32),
            pltpu.VMEM((nb, BLK), jnp.int32),
            pltpu.VMEM((BLK, row_s), jnp.float32),
            pltpu.VMEM((BLK, 16), jnp.float32),
            pltpu.VMEM((BLK, row_o), jnp.float32),
            pltpu.VMEM_SHARED((N1, row_o), jnp.float32),
            pltpu.SemaphoreType.DMA,
            pltpu.SemaphoreType.DMA,
        ],
    )
    def k(tsrc_hbm, tdst_hbm, sidx_hbm, didx_hbm, out_hbm,
          sidx_v, didx_v, srows, drows, orows, acc, sem1, sem2):
        c = lax.axis_index("c")
        s = lax.axis_index("s")
        wid = c * NSUB + s

        pltpu.sync_copy(sidx_hbm.at[wid], sidx_v)
        pltpu.sync_copy(didx_hbm.at[wid], didx_v)

        zero = jnp.zeros((16,), jnp.float32)

        def zrow(e, _):
            for col in range(row_o // 16):
                orows[e, pl.ds(16 * col, 16)] = zero
            return 0

        lax.fori_loop(0, zchunk, zrow, 0)
        base = s * rows_per_sub

        def zcopy(i, _):
            pltpu.sync_copy(orows,
                            acc.at[pl.ds(base + i * zchunk, zchunk)])
            return 0

        lax.fori_loop(0, rows_per_sub // zchunk, zcopy, 0)
        plsc.subcore_barrier()

        def block(j, _):
            cp1 = pltpu.async_copy(tsrc_hbm.at[sidx_v.at[j]], srows, sem1)
            cp2 = pltpu.async_copy(tdst_hbm.at[didx_v.at[j]], drows, sem2)
            cp1.wait()
            cp2.wait()

            def edge(e, _):
                edge_fn(srows, drows, orows, e)
                return 0

            lax.fori_loop(0, BLK, edge, 0)
            pltpu.sync_copy(orows, acc.at[didx_v.at[j]], add=True)
            return 0

        lax.fori_loop(0, nb, block, 0)
        plsc.subcore_barrier()

        pltpu.sync_copy(acc.at[pl.ds(base, rows_per_sub)],
                        out_hbm.at[c, pl.ds(base, rows_per_sub)])

    return k(tsrc, tdst, src_idx, dst_idx)


def kernel(high_dim_features, low_dim_features, edge_index, W_emb, b_emb, W1,
           att_src1, att_dst1, bias1, W2, att_src2, att_dst2, bias2):
    n, high = high_dim_features.shape
    e = edge_index.shape[1]
    et = e + n
    nb = -(-et // (NTILES * BLK))
    et_pad = NTILES * BLK * nb

    loop = jnp.arange(n, dtype=jnp.int32)
    src = jnp.concatenate([edge_index[0].astype(jnp.int32), loop])
    dst = jnp.concatenate([edge_index[1].astype(jnp.int32), loop])
    pad = jnp.full((et_pad - et,), n, jnp.int32)
    src = jnp.concatenate([src, pad]).reshape(NTILES, nb, BLK)
    dst = jnp.concatenate([dst, pad]).reshape(NTILES, nb, BLK)

    high_p = jnp.pad(high_dim_features, ((0, N1 - n), (0, 0)))
    low_p = jnp.pad(low_dim_features, ((0, N1 - n), (0, 0)))

    heads, hid = att_src1.shape[1], att_src1.shape[2]
    eye = jnp.eye(heads, dtype=jnp.float32)
    s8 = (att_src1[0][:, :, None] * eye[:, None, :]).reshape(heads * hid, heads)
    d8 = (att_dst1[0][:, :, None] * eye[:, None, :]).reshape(heads * hid, heads)
    S2 = jnp.concatenate([s8, s8], axis=1)
    D2 = jnp.concatenate([d8, d8], axis=1)

    r8 = jnp.kron(jnp.eye(heads, dtype=jnp.float32),
                  jnp.ones((1, hid), jnp.float32))
    R = jnp.concatenate([r8, jnp.zeros_like(r8)], axis=0)
    a2s = jnp.tile(att_src2[0].reshape(-1, 1), (1, 16))
    a2d = jnp.tile(att_dst2[0].reshape(-1, 1), (1, 16))

    tsrc1, tdst1 = _tc_prologue(high_p, low_p, W_emb, b_emb.reshape(1, -1),
                                W1[:high], W1[high:], S2, D2)
    acc1 = _sc_edge_pass(tsrc1, tdst1, src, dst, nb, 80, 80, _edge_compute_l1)
    tsrc2, tdst2 = _tc_mid(acc1[0], acc1[1], R, bias1.reshape(1, -1), W2,
                           a2s, a2d)
    acc2 = _sc_edge_pass(tsrc2, tdst2, src, dst, nb, 32, 32, _edge_compute_l2)
    out = _tc_epilogue(acc2[0], acc2[1], bias2.reshape(1, -1))
    return out[:n]
